# baseline (speedup 1.0000x reference)
import functools
import jax
import numpy as np
import jax.numpy as jnp
from jax import lax
from jax.experimental import pallas as pl
from jax.experimental.pallas import tpu as pltpu
from jax.experimental.pallas import tpu_sc as plsc

B = 16384
H = 64

_info = plsc.get_sparse_core_info()
NC = _info.num_cores
NS = _info.num_subcores
L = _info.num_lanes
NW = NC * NS
RPW = B // NW
IC = 128
NIC = RPW // IC
GB = 2

_mesh = plsc.VectorSubcoreMesh(core_axis_name="c", subcore_axis_name="s")


@functools.partial(
    pl.kernel,
    mesh=_mesh,
    out_type=jax.ShapeDtypeStruct((B,), jnp.float32),
    compiler_params=pltpu.CompilerParams(needs_layout_passes=False,
                                         use_tc_tiling_on_sc=True),
    scratch_types=[
        pltpu.VMEM((NIC, IC), jnp.int32),
        pltpu.VMEM((NIC, IC), jnp.int32),
        pltpu.VMEM((RPW,), jnp.float32),
        pltpu.VMEM((RPW,), jnp.float32),
        pltpu.VMEM((H, RPW), jnp.float32),
        pltpu.VMEM((H, RPW), jnp.float32),
        pltpu.VMEM((2, IC), jnp.int32),
        pltpu.VMEM((IC,), jnp.float32),
        pltpu.VMEM((IC,), jnp.float32),
        pltpu.VMEM((RPW,), jnp.float32),
        pltpu.SemaphoreType.DMA,
        pltpu.SemaphoreType.DMA,
        pltpu.SemaphoreType.DMA,
        pltpu.SemaphoreType.DMA,
        pltpu.SemaphoreType.DMA,
        pltpu.SemaphoreType.DMA,
    ],
)
def _lf_kernel(uf_hbm, uid_hbm, if_hbm, iid_hbm, w_hbm, b_hbm, cidx_hbm,
               bu_hbm, bi_hbm,
               out_hbm, uidx_v, iidx_v, ub_v, ib_v, uf_v, if_v, cidx_v, w_v,
               b_v, out_v, sem_i, sem_g, sem_f0, sem_f1, sem_f2, sem_f3):
    wid = lax.axis_index("s") * NC + lax.axis_index("c")
    col0 = wid * RPW
    ic0 = wid * NIC
    sem_f = [sem_f0, sem_f1, sem_f2, sem_f3]

    pltpu.sync_copy(uid_hbm.at[pl.ds(ic0, NIC)], uidx_v)
    pltpu.sync_copy(iid_hbm.at[pl.ds(ic0, NIC)], iidx_v)
    pltpu.sync_copy(cidx_hbm, cidx_v)

    cf1 = pltpu.async_copy(uf_hbm.at[:, pl.ds(col0, RPW)], uf_v, sem_f0)
    cf2 = pltpu.async_copy(if_hbm.at[:, pl.ds(col0, RPW)], if_v, sem_f0)

    gathers = []
    for j in range(NIC):
        gathers.append(
            pltpu.async_copy(bu_hbm.at[uidx_v.at[j]],
                             ub_v.at[pl.ds(j * IC, IC)], sem_g))
        gathers.append(
            pltpu.async_copy(bi_hbm.at[iidx_v.at[j]],
                             ib_v.at[pl.ds(j * IC, IC)], sem_g))

    gw = pltpu.async_copy(w_hbm.at[cidx_v.at[0]], w_v, sem_g)
    gb = pltpu.async_copy(b_hbm.at[cidx_v.at[1]], b_v, sem_g)

    for g in gathers:
        g.wait()
    gw.wait()
    gb.wait()
    cf1.wait()
    cf2.wait()

    bv = b_v[pl.ds(0, L)]

    def groupbody(gg, carry):
        base = pl.multiple_of(gg * (GB * L), GB * L)
        accs = [None] * GB
        for h in range(H):
            wbh = plsc.load_gather(w_v, [jnp.full((L,), h, jnp.int32)])
            for q in range(GB):
                prod = (uf_v[h, pl.ds(base + q * L, L)]
                        * if_v[h, pl.ds(base + q * L, L)] * wbh)
                accs[q] = prod if h == 0 else accs[q] + prod
        for q in range(GB):
            o = pl.ds(base + q * L, L)
            out_v[o] = accs[q] + ub_v[o] + ib_v[o] + bv
        return carry

    lax.fori_loop(0, RPW // (GB * L), groupbody, 0)

    pltpu.sync_copy(out_v, out_hbm.at[pl.ds(col0, RPW)])


def kernel(user_feature, user_id, item_feature, item_id, W, b, b_user, b_item):
    uft = user_feature.T
    ift = item_feature.T
    uid = user_id.reshape(B // IC, IC)
    iid = item_id.reshape(B // IC, IC)
    w = W.reshape(H)
    cidx = jnp.asarray(np.stack([
        np.pad(np.arange(H, dtype=np.int32), (0, IC - H)),
        np.zeros(IC, dtype=np.int32),
    ]))
    out = _lf_kernel(uft, uid, ift, iid, w, b, cidx, b_user, b_item)
    return out.reshape(B, 1)


# speedup vs baseline: 2.1573x; 2.1573x over previous
import functools
import jax
import jax.numpy as jnp
from jax import lax
from jax.experimental import pallas as pl
from jax.experimental.pallas import tpu as pltpu
from jax.experimental.pallas import tpu_sc as plsc

B = 16384
H = 64

_info = plsc.get_sparse_core_info()
NC = _info.num_cores
NS = _info.num_subcores
L = _info.num_lanes
NW = NC * NS
RPW = B // NW
IC = 128
NIC = RPW // IC

_mesh = plsc.VectorSubcoreMesh(core_axis_name="c", subcore_axis_name="s")


@functools.partial(
    pl.kernel,
    mesh=_mesh,
    out_type=jax.ShapeDtypeStruct((B,), jnp.float32),
    compiler_params=pltpu.CompilerParams(needs_layout_passes=False,
                                         use_tc_tiling_on_sc=True),
    scratch_types=[
        pltpu.VMEM((NIC, IC), jnp.int32),
        pltpu.VMEM((NIC, IC), jnp.int32),
        pltpu.VMEM((RPW,), jnp.float32),
        pltpu.VMEM((RPW,), jnp.float32),
        pltpu.VMEM((RPW,), jnp.float32),
        pltpu.SemaphoreType.DMA,
    ],
)
def _bias_kernel(uid_hbm, iid_hbm, bu_hbm, bi_hbm, out_hbm,
                 uidx_v, iidx_v, ub_v, ib_v, out_v, sem_g):
    wid = lax.axis_index("s") * NC + lax.axis_index("c")
    col0 = wid * RPW
    ic0 = wid * NIC

    pltpu.sync_copy(uid_hbm.at[pl.ds(ic0, NIC)], uidx_v)
    pltpu.sync_copy(iid_hbm.at[pl.ds(ic0, NIC)], iidx_v)

    gathers = []
    for j in range(NIC):
        gathers.append(
            pltpu.async_copy(bu_hbm.at[uidx_v.at[j]],
                             ub_v.at[pl.ds(j * IC, IC)], sem_g))
        gathers.append(
            pltpu.async_copy(bi_hbm.at[iidx_v.at[j]],
                             ib_v.at[pl.ds(j * IC, IC)], sem_g))
    for g in gathers:
        g.wait()

    def addbody(c, carry):
        o = pl.ds(pl.multiple_of(c * L, L), L)
        out_v[o] = ub_v[o] + ib_v[o]
        return carry

    lax.fori_loop(0, RPW // L, addbody, 0)

    pltpu.sync_copy(out_v, out_hbm.at[pl.ds(col0, RPW)])


_TCB = 2048


def _dots_body(uf_ref, if_ref, w_ref, b_ref, out_ref):
    prod = uf_ref[...] * if_ref[...] * w_ref[...]
    out_ref[...] = jnp.sum(prod, axis=0) + b_ref[0, 0]


_dots_kernel = pl.pallas_call(
    _dots_body,
    grid=(B // _TCB,),
    in_specs=[
        pl.BlockSpec((H, _TCB), lambda j: (0, j)),
        pl.BlockSpec((H, _TCB), lambda j: (0, j)),
        pl.BlockSpec((H, 1), lambda j: (0, 0)),
        pl.BlockSpec((1, 1), lambda j: (0, 0)),
    ],
    out_specs=pl.BlockSpec((_TCB,), lambda j: (j,)),
    out_shape=jax.ShapeDtypeStruct((B,), jnp.float32),
)


def kernel(user_feature, user_id, item_feature, item_id, W, b, b_user, b_item):
    uft = user_feature.T
    ift = item_feature.T
    uid = user_id.reshape(B // IC, IC)
    iid = item_id.reshape(B // IC, IC)
    scb = _bias_kernel(uid, iid, b_user, b_item)
    dots = _dots_kernel(uft, ift, W, b.reshape(1, 1))
    return (dots + scb).reshape(B, 1)


# speedup vs baseline: 2.2319x; 1.0346x over previous
import functools
import jax
import jax.numpy as jnp
from jax import lax
from jax.experimental import pallas as pl
from jax.experimental.pallas import tpu as pltpu
from jax.experimental.pallas import tpu_sc as plsc

B = 16384
H = 64

_info = plsc.get_sparse_core_info()
NC = _info.num_cores
NS = _info.num_subcores
L = _info.num_lanes
NW = NC * NS
RPW = B // NW
IC = 128
NIC = RPW // IC

_mesh = plsc.VectorSubcoreMesh(core_axis_name="c", subcore_axis_name="s")


@functools.partial(
    pl.kernel,
    mesh=_mesh,
    out_type=jax.ShapeDtypeStruct((B,), jnp.float32),
    compiler_params=pltpu.CompilerParams(needs_layout_passes=False,
                                         use_tc_tiling_on_sc=True),
    scratch_types=[
        pltpu.VMEM((NIC, IC), jnp.int32),
        pltpu.VMEM((NIC, IC), jnp.int32),
        pltpu.VMEM((RPW,), jnp.float32),
        pltpu.VMEM((RPW,), jnp.float32),
        pltpu.VMEM((RPW,), jnp.float32),
        pltpu.SemaphoreType.DMA,
    ],
)
def _bias_kernel(uid_hbm, iid_hbm, bu_hbm, bi_hbm, out_hbm,
                 uidx_v, iidx_v, ub_v, ib_v, out_v, sem_g):
    wid = lax.axis_index("s") * NC + lax.axis_index("c")
    col0 = wid * RPW
    ic0 = wid * NIC

    pltpu.sync_copy(uid_hbm.at[pl.ds(ic0, NIC)], uidx_v)
    pltpu.sync_copy(iid_hbm.at[pl.ds(ic0, NIC)], iidx_v)

    gathers = []
    for j in range(NIC):
        gathers.append(
            pltpu.async_copy(bu_hbm.at[uidx_v.at[j]],
                             ub_v.at[pl.ds(j * IC, IC)], sem_g))
        gathers.append(
            pltpu.async_copy(bi_hbm.at[iidx_v.at[j]],
                             ib_v.at[pl.ds(j * IC, IC)], sem_g))
    for g in gathers:
        g.wait()

    def addbody(c, carry):
        o = pl.ds(pl.multiple_of(c * L, L), L)
        out_v[o] = ub_v[o] + ib_v[o]
        return carry

    lax.fori_loop(0, RPW // L, addbody, 0)

    pltpu.sync_copy(out_v, out_hbm.at[pl.ds(col0, RPW)])


_TCB = 4096


def _dots_body(uf_ref, if_ref, w_ref, b_ref, out_ref):
    prod = uf_ref[...] * if_ref[...] * w_ref[...]
    out_ref[...] = jnp.sum(prod, axis=0) + b_ref[0, 0]


_dots_kernel = pl.pallas_call(
    _dots_body,
    grid=(B // _TCB,),
    in_specs=[
        pl.BlockSpec((H, _TCB), lambda j: (0, j)),
        pl.BlockSpec((H, _TCB), lambda j: (0, j)),
        pl.BlockSpec((H, 1), lambda j: (0, 0)),
        pl.BlockSpec((1, 1), lambda j: (0, 0)),
    ],
    out_specs=pl.BlockSpec((_TCB,), lambda j: (j,)),
    out_shape=jax.ShapeDtypeStruct((B,), jnp.float32),
)


def kernel(user_feature, user_id, item_feature, item_id, W, b, b_user, b_item):
    uft = user_feature.T
    ift = item_feature.T
    uid = user_id.reshape(B // IC, IC)
    iid = item_id.reshape(B // IC, IC)
    scb = _bias_kernel(uid, iid, b_user, b_item)
    dots = _dots_kernel(uft, ift, W, b.reshape(1, 1))
    return (dots + scb).reshape(B, 1)
